# baseline (speedup 1.0000x reference)
import functools

import jax
import jax.numpy as jnp
from jax import lax
from jax.experimental import pallas as pl
from jax.experimental.pallas import tpu as pltpu
from jax.experimental.pallas import tpu_sc as plsc

_ROWS = 100000
_KD = 128
_H2 = 64
_CH = 128
_NCH = 49
_TILE_ROWS = 6256
_CLAMP = _ROWS - _CH
_NBUF = 3
_BIG = 3.0e38
_IMAX = 2**31 - 1


def _vgather(vec, idx):
  dnums = lax.GatherDimensionNumbers(
      offset_dims=(), collapsed_slice_dims=(0,), start_index_map=(0,))
  return lax.gather(vec, idx[:, None], dnums, (1,),
                    mode=lax.GatherScatterMode.PROMISE_IN_BOUNDS)


def _sc_retrieve(cue, keys1, vals1, keys2, vals2):
  mesh = plsc.VectorSubcoreMesh(core_axis_name="c", subcore_axis_name="s",
                                num_cores=2, num_subcores=16)

  def body(cue_hbm, keys1_hbm, vals1_hbm, keys2_hbm, vals2_hbm,
           m1_hbm, m2_hbm,
           b0, b1, b2, cue_v, stage_d, stage_i, alld, alli, idxv,
           row1, row2, shd, shi, sem0, sem1, sem2, gsem):
    cid = lax.axis_index("c")
    tid = lax.axis_index("s")
    iota16 = lax.iota(jnp.int32, 16)
    bufs = (b0, b1, b2)
    sems = (sem0, sem1, sem2)

    pltpu.sync_copy(cue_hbm.at[0], cue_v)
    for i in range(8):
      sl = pl.ds(i * 16, 16)
      cue_v[sl] = cue_v[sl] * 2.0

    base = tid * _TILE_ROWS
    flatbase_g = tuple(iota16 * _KD + g * 16 * _KD for g in range(8))

    def scan_dict(keys_hbm, vals_hbm, rowbuf, out_hbm, half):
      def start_chunk(k, b):
        st = jnp.minimum(base + k * _CH, _CLAMP)
        pltpu.make_async_copy(
            keys_hbm.at[pl.ds(st * _KD, _CH * _KD)], bufs[b], sems[b]).start()

      def wait_chunk(b):
        pltpu.make_async_copy(
            keys_hbm.at[pl.ds(0, _CH * _KD)], bufs[b], sems[b]).wait()

      for b in range(_NBUF):
        start_chunk(jnp.int32(b), b)

      def do_chunk(k, b, minv, mini):
        wait_chunk(b)
        st = jnp.minimum(base + k * _CH, _CLAMP)
        accs = tuple(jnp.zeros((16,), jnp.float32) for _ in range(8))
        for jb in range(8):
          cvec = cue_v[pl.ds(jb * 16, 16)]

          def jj_body(jj, accs_t, jb=jb, cvec=cvec, b=b):
            c2 = _vgather(cvec, jnp.full((16,), jj, jnp.int32))
            col = jnp.full((16,), jb * 16 + jj, jnp.int32)
            out = []
            for g in range(8):
              kv = plsc.load_gather(bufs[b], (flatbase_g[g] + col,))
              out.append(accs_t[g] + kv * (kv - c2))
            return tuple(out)

          accs = lax.fori_loop(0, 16, jj_body, accs)
        for g in range(8):
          rows = st + g * 16 + iota16
          d = accs[g]
          better = (d < minv) | ((d == minv) & (rows < mini))
          minv = jnp.where(better, d, minv)
          mini = jnp.where(better, rows, mini)
        return minv, mini

      minv = jnp.full((16,), _BIG, jnp.float32)
      mini = jnp.zeros((16,), jnp.int32)

      def outer(c, carry):
        mv, mi = carry
        for b in range(_NBUF):
          k = c * _NBUF + b
          mv, mi = do_chunk(k, b, mv, mi)

          @pl.when(k + _NBUF < _NCH)
          def _(k=k, b=b):
            start_chunk(k + _NBUF, b)
        return mv, mi

      minv, mini = lax.fori_loop(0, (_NCH - 1) // _NBUF, outer, (minv, mini))
      minv, mini = do_chunk(jnp.int32(_NCH - 1), 0, minv, mini)

      stage_d[...] = minv
      stage_i[...] = mini
      pltpu.sync_copy(stage_d, shd.at[pl.ds(tid * 16, 16)])
      pltpu.sync_copy(stage_i, shi.at[pl.ds(tid * 16, 16)])
      plsc.subcore_barrier()

      @pl.when(tid == 0)
      def _():
        pltpu.sync_copy(shd, alld)
        pltpu.sync_copy(shi, alli)
        mv = jnp.full((16,), _BIG, jnp.float32)
        mi = jnp.full((16,), _IMAX, jnp.int32)
        for t in range(16):
          d = alld[pl.ds(t * 16, 16)]
          ii = alli[pl.ds(t * 16, 16)]
          better = (d < mv) | ((d == mv) & (ii < mi))
          mv = jnp.where(better, d, mv)
          mi = jnp.where(better, ii, mi)
        m = jnp.min(mv)
        cand = jnp.where(mv == m, mi, jnp.full((16,), _IMAX, jnp.int32))
        bi = jnp.min(cand)
        if half:
          idxv[...] = jnp.full((16,), bi >> 1, jnp.int32)
        else:
          idxv[...] = jnp.full((16,), bi, jnp.int32)
        pltpu.async_copy(vals_hbm.at[idxv], rowbuf, gsem).wait()
        if half:
          off = (bi & 1) * _H2
          pltpu.sync_copy(rowbuf.at[0, pl.ds(off, _H2)], out_hbm.at[0])
        else:
          pltpu.sync_copy(rowbuf.at[pl.ds(0, 1)], out_hbm)

    @pl.when(cid == 0)
    def _():
      scan_dict(keys1_hbm, vals1_hbm, row1, m1_hbm, half=False)

    @pl.when(cid == 1)
    def _():
      scan_dict(keys2_hbm, vals2_hbm, row2, m2_hbm, half=True)

  f = pl.kernel(
      body,
      out_type=(jax.ShapeDtypeStruct((1, _KD), jnp.float32),
                jax.ShapeDtypeStruct((1, _H2), jnp.float32)),
      mesh=mesh,
      compiler_params=pltpu.CompilerParams(needs_layout_passes=False),
      scratch_types=[
          pltpu.VMEM((_CH * _KD,), jnp.float32),
          pltpu.VMEM((_CH * _KD,), jnp.float32),
          pltpu.VMEM((_CH * _KD,), jnp.float32),
          pltpu.VMEM((_KD,), jnp.float32),
          pltpu.VMEM((16,), jnp.float32),
          pltpu.VMEM((16,), jnp.int32),
          pltpu.VMEM((256,), jnp.float32),
          pltpu.VMEM((256,), jnp.int32),
          pltpu.VMEM((16,), jnp.int32),
          pltpu.VMEM((16, _KD), jnp.float32),
          pltpu.VMEM((16, _KD), jnp.float32),
          pltpu.VMEM_SHARED((256,), jnp.float32),
          pltpu.VMEM_SHARED((256,), jnp.int32),
          pltpu.SemaphoreType.DMA,
          pltpu.SemaphoreType.DMA,
          pltpu.SemaphoreType.DMA,
          pltpu.SemaphoreType.DMA,
      ],
  )
  return f(cue, keys1.reshape(-1), vals1, keys2.reshape(-1),
           vals2.reshape(_ROWS // 2, _KD))


def _sig(x):
  return 1.0 / (1.0 + jnp.exp(-x))


def _dense_body(obs_r, pa_r, pr_r, h1_r, c1_r, h2_r, c2_r,
                w1_r, be1_r, w2_r, be2_r,
                wih1_r, whh1_r, bi1_r, bh1_r,
                wih2_r, whh2_r, bi2_r, bh2_r,
                aw_r, ab_r, cw_r, cb_r, m1_r, m2_r,
                lo_r, vo_r, h1o_r, c1o_r, h2o_r, c2o_r):
  def mmT(x, w):
    return lax.dot_general(x, w, (((1,), (1,)), ((), ())),
                           preferred_element_type=jnp.float32)

  obs_v = obs_r[...]
  f1 = jnp.maximum(mmT(obs_v, w1_r[...]) + be1_r[...], 0.0)
  feats = jnp.maximum(mmT(f1, w2_r[...]) + be2_r[...], 0.0)

  wih1 = wih1_r[...]
  h1v = h1_r[...]
  c1v = c1_r[...]
  g = (mmT(feats, wih1[:, 0:128]) + mmT(pr_r[...], wih1[:, 128:129]) +
       mmT(h1v, whh1_r[...]) + bi1_r[...] + bh1_r[...])
  i_g = g[:, 0:128]
  f_g = g[:, 128:256]
  g_g = g[:, 256:384]
  o_g = g[:, 384:512]
  r_g = g[:, 512:640]
  c1n = _sig(f_g) * c1v + _sig(i_g) * jnp.tanh(g_g) + _sig(r_g) * m1_r[...]
  h1n = _sig(o_g) * jnp.tanh(c1n)

  wih2 = wih2_r[...]
  h2v = h2_r[...]
  c2v = c2_r[...]
  g2 = (mmT(h1n, wih2[:, 0:128]) + mmT(feats, wih2[:, 128:256]) +
        mmT(pa_r[...], wih2[:, 256:262]) + mmT(h2v, whh2_r[...]) +
        bi2_r[...] + bh2_r[...])
  i2 = g2[:, 0:64]
  f2 = g2[:, 64:128]
  gg2 = g2[:, 128:192]
  o2 = g2[:, 192:256]
  r2 = g2[:, 256:320]
  c2n = _sig(f2) * c2v + _sig(i2) * jnp.tanh(gg2) + _sig(r2) * m2_r[...]
  h2n = _sig(o2) * jnp.tanh(c2n)

  lo_r[...] = mmT(h2n, aw_r[...]) + ab_r[...]
  vo_r[...] = jnp.sum(h2n * cw_r[...], axis=1, keepdims=True) + cb_r[...]
  h1o_r[...] = h1n
  c1o_r[...] = c1n
  h2o_r[...] = h2n
  c2o_r[...] = c2n


def _dense(obs, p_action, p_reward, h1, c1, h2, c2,
           enc_W1, enc_b1, enc_W2, enc_b2,
           Wih1, Whh1, bih1, bhh1, Wih2, Whh2, bih2, bhh2,
           actor_W, actor_b, critic_W, critic_b, m1, m2):
  out_shape = (
      jax.ShapeDtypeStruct((1, 6), jnp.float32),
      jax.ShapeDtypeStruct((1, 1), jnp.float32),
      jax.ShapeDtypeStruct((1, 128), jnp.float32),
      jax.ShapeDtypeStruct((1, 128), jnp.float32),
      jax.ShapeDtypeStruct((1, 64), jnp.float32),
      jax.ShapeDtypeStruct((1, 64), jnp.float32),
  )
  return pl.pallas_call(_dense_body, out_shape=out_shape)(
      obs, p_action, p_reward, h1, c1, h2, c2,
      enc_W1, enc_b1.reshape(1, -1), enc_W2, enc_b2.reshape(1, -1),
      Wih1, Whh1, bih1.reshape(1, -1), bhh1.reshape(1, -1),
      Wih2, Whh2, bih2.reshape(1, -1), bhh2.reshape(1, -1),
      actor_W, actor_b.reshape(1, -1), critic_W, critic_b.reshape(1, -1),
      m1, m2)


def kernel(obs, p_action, p_reward, h1, c1, h2, c2, cue,
           enc_W1, enc_b1, enc_W2, enc_b2, keys1, vals1, keys2, vals2,
           Wih1, Whh1, bih1, bhh1, Wih2, Whh2, bih2, bhh2,
           actor_W, actor_b, critic_W, critic_b):
  m1, m2 = _sc_retrieve(cue, keys1, vals1, keys2, vals2)
  lo, vo, h1o, c1o, h2o, c2o = _dense(
      obs, p_action, p_reward, h1[0], c1[0], h2[0], c2[0],
      enc_W1, enc_b1, enc_W2, enc_b2,
      Wih1, Whh1, bih1, bhh1, Wih2, Whh2, bih2, bhh2,
      actor_W, actor_b, critic_W, critic_b, m1, m2)
  return (lo[:, None, :], vo[:, None, :], h1o[None], c1o[None],
          h2o[None], c2o[None])


# speedup vs baseline: 3.6972x; 3.6972x over previous
import functools

import jax
import jax.numpy as jnp
from jax import lax
from jax.experimental import pallas as pl
from jax.experimental.pallas import tpu as pltpu
from jax.experimental.pallas import tpu_sc as plsc

_ROWS = 100000
_KD = 128
_H2 = 64
_CH = 128
_NCH = 49
_TILE_ROWS = 6256
_CLAMP = _ROWS - _CH
_NBUF = 3
_BIG = 3.0e38
_IMAX = 2**31 - 1


def _vgather(vec, idx):
  dnums = lax.GatherDimensionNumbers(
      offset_dims=(), collapsed_slice_dims=(0,), start_index_map=(0,))
  return lax.gather(vec, idx[:, None], dnums, (1,),
                    mode=lax.GatherScatterMode.PROMISE_IN_BOUNDS)


def _sc_retrieve(cue, keys1, vals1, keys2, vals2):
  mesh = plsc.VectorSubcoreMesh(core_axis_name="c", subcore_axis_name="s",
                                num_cores=2, num_subcores=16)

  def body(cue_hbm, keys1_hbm, vals1_hbm, keys2_hbm, vals2_hbm,
           m1_hbm, m2_hbm,
           b0, b1, b2, cue_v, stage_d, stage_i, alld, alli, idxv,
           row1, row2, shd, shi, sem0, sem1, sem2, gsem):
    cid = lax.axis_index("c")
    tid = lax.axis_index("s")
    iota16 = lax.iota(jnp.int32, 16)
    bufs = (b0, b1, b2)
    sems = (sem0, sem1, sem2)

    pltpu.sync_copy(cue_hbm.at[0], cue_v.at[pl.ds(0, _KD)])
    for i in range(8):
      sl = pl.ds(i * 16, 16)
      cue_v[sl] = cue_v[sl] * 2.0
    cue_v[pl.ds(_KD, 16)] = cue_v[pl.ds(0, 16)]

    base = tid * _TILE_ROWS
    rowoff_g = tuple(iota16 * _KD + g * 16 * _KD for g in range(8))

    def scan_dict(keys_hbm, vals_hbm, rowbuf, out_hbm, half):
      def start_chunk(k, b):
        st = jnp.minimum(base + k * _CH, _CLAMP)
        pltpu.make_async_copy(
            keys_hbm.at[pl.ds(st * _KD, _CH * _KD)], bufs[b], sems[b]).start()

      def wait_chunk(b):
        pltpu.make_async_copy(
            keys_hbm.at[pl.ds(0, _CH * _KD)], bufs[b], sems[b]).wait()

      for b in range(_NBUF):
        start_chunk(jnp.int32(b), b)

      def do_chunk(k, b, minv, mini):
        wait_chunk(b)
        st = jnp.minimum(base + k * _CH, _CLAMP)
        accs = tuple(jnp.zeros((16,), jnp.float32) for _ in range(8))

        def s_body(s, carry, b=b):
          jpos = (iota16 + s) & (_KD - 1)
          c2 = cue_v[pl.ds(s, 16)]
          accs_t = carry
          out = []
          for g in range(8):
            kv = plsc.load_gather(bufs[b], (rowoff_g[g] + jpos,))
            out.append(accs_t[g] + kv * (kv - c2))
          return tuple(out)

        accs = lax.fori_loop(0, _KD, s_body, accs)
        for g in range(8):
          rows = st + g * 16 + iota16
          d = accs[g]
          better = (d < minv) | ((d == minv) & (rows < mini))
          minv = jnp.where(better, d, minv)
          mini = jnp.where(better, rows, mini)
        return minv, mini

      minv = jnp.full((16,), _BIG, jnp.float32)
      mini = jnp.zeros((16,), jnp.int32)

      def outer(c, carry):
        mv, mi = carry
        for b in range(_NBUF):
          k = c * _NBUF + b
          mv, mi = do_chunk(k, b, mv, mi)

          @pl.when(k + _NBUF < _NCH)
          def _(k=k, b=b):
            start_chunk(k + _NBUF, b)
        return mv, mi

      minv, mini = lax.fori_loop(0, (_NCH - 1) // _NBUF, outer, (minv, mini))
      minv, mini = do_chunk(jnp.int32(_NCH - 1), 0, minv, mini)

      stage_d[...] = minv
      stage_i[...] = mini
      pltpu.sync_copy(stage_d, shd.at[pl.ds(tid * 16, 16)])
      pltpu.sync_copy(stage_i, shi.at[pl.ds(tid * 16, 16)])
      plsc.subcore_barrier()

      @pl.when(tid == 0)
      def _():
        pltpu.sync_copy(shd, alld)
        pltpu.sync_copy(shi, alli)
        mv = jnp.full((16,), _BIG, jnp.float32)
        mi = jnp.full((16,), _IMAX, jnp.int32)
        for t in range(16):
          d = alld[pl.ds(t * 16, 16)]
          ii = alli[pl.ds(t * 16, 16)]
          better = (d < mv) | ((d == mv) & (ii < mi))
          mv = jnp.where(better, d, mv)
          mi = jnp.where(better, ii, mi)
        m = jnp.min(mv)
        cand = jnp.where(mv == m, mi, jnp.full((16,), _IMAX, jnp.int32))
        bi = jnp.min(cand)
        if half:
          idxv[...] = jnp.full((16,), bi >> 1, jnp.int32)
        else:
          idxv[...] = jnp.full((16,), bi, jnp.int32)
        pltpu.async_copy(vals_hbm.at[idxv], rowbuf, gsem).wait()
        if half:
          off = (bi & 1) * _H2
          pltpu.sync_copy(rowbuf.at[0, pl.ds(off, _H2)], out_hbm.at[0])
        else:
          pltpu.sync_copy(rowbuf.at[pl.ds(0, 1)], out_hbm)

    @pl.when(cid == 0)
    def _():
      scan_dict(keys1_hbm, vals1_hbm, row1, m1_hbm, half=False)

    @pl.when(cid == 1)
    def _():
      scan_dict(keys2_hbm, vals2_hbm, row2, m2_hbm, half=True)

  f = pl.kernel(
      body,
      out_type=(jax.ShapeDtypeStruct((1, _KD), jnp.float32),
                jax.ShapeDtypeStruct((1, _H2), jnp.float32)),
      mesh=mesh,
      compiler_params=pltpu.CompilerParams(needs_layout_passes=False),
      scratch_types=[
          pltpu.VMEM((_CH * _KD,), jnp.float32),
          pltpu.VMEM((_CH * _KD,), jnp.float32),
          pltpu.VMEM((_CH * _KD,), jnp.float32),
          pltpu.VMEM((_KD + 16,), jnp.float32),
          pltpu.VMEM((16,), jnp.float32),
          pltpu.VMEM((16,), jnp.int32),
          pltpu.VMEM((256,), jnp.float32),
          pltpu.VMEM((256,), jnp.int32),
          pltpu.VMEM((16,), jnp.int32),
          pltpu.VMEM((16, _KD), jnp.float32),
          pltpu.VMEM((16, _KD), jnp.float32),
          pltpu.VMEM_SHARED((256,), jnp.float32),
          pltpu.VMEM_SHARED((256,), jnp.int32),
          pltpu.SemaphoreType.DMA,
          pltpu.SemaphoreType.DMA,
          pltpu.SemaphoreType.DMA,
          pltpu.SemaphoreType.DMA,
      ],
  )
  return f(cue, keys1.reshape(-1), vals1, keys2.reshape(-1),
           vals2.reshape(_ROWS // 2, _KD))


def _sig(x):
  return 1.0 / (1.0 + jnp.exp(-x))


def _dense_body(obs_r, pa_r, pr_r, h1_r, c1_r, h2_r, c2_r,
                w1_r, be1_r, w2_r, be2_r,
                wih1_r, whh1_r, bi1_r, bh1_r,
                wih2_r, whh2_r, bi2_r, bh2_r,
                aw_r, ab_r, cw_r, cb_r, m1_r, m2_r,
                lo_r, vo_r, h1o_r, c1o_r, h2o_r, c2o_r):
  def mmT(x, w):
    return lax.dot_general(x, w, (((1,), (1,)), ((), ())),
                           preferred_element_type=jnp.float32)

  obs_v = obs_r[...]
  f1 = jnp.maximum(mmT(obs_v, w1_r[...]) + be1_r[...], 0.0)
  feats = jnp.maximum(mmT(f1, w2_r[...]) + be2_r[...], 0.0)

  wih1 = wih1_r[...]
  h1v = h1_r[...]
  c1v = c1_r[...]
  g = (mmT(feats, wih1[:, 0:128]) + mmT(pr_r[...], wih1[:, 128:129]) +
       mmT(h1v, whh1_r[...]) + bi1_r[...] + bh1_r[...])
  i_g = g[:, 0:128]
  f_g = g[:, 128:256]
  g_g = g[:, 256:384]
  o_g = g[:, 384:512]
  r_g = g[:, 512:640]
  c1n = _sig(f_g) * c1v + _sig(i_g) * jnp.tanh(g_g) + _sig(r_g) * m1_r[...]
  h1n = _sig(o_g) * jnp.tanh(c1n)

  wih2 = wih2_r[...]
  h2v = h2_r[...]
  c2v = c2_r[...]
  g2 = (mmT(h1n, wih2[:, 0:128]) + mmT(feats, wih2[:, 128:256]) +
        mmT(pa_r[...], wih2[:, 256:262]) + mmT(h2v, whh2_r[...]) +
        bi2_r[...] + bh2_r[...])
  i2 = g2[:, 0:64]
  f2 = g2[:, 64:128]
  gg2 = g2[:, 128:192]
  o2 = g2[:, 192:256]
  r2 = g2[:, 256:320]
  c2n = _sig(f2) * c2v + _sig(i2) * jnp.tanh(gg2) + _sig(r2) * m2_r[...]
  h2n = _sig(o2) * jnp.tanh(c2n)

  lo_r[...] = mmT(h2n, aw_r[...]) + ab_r[...]
  vo_r[...] = jnp.sum(h2n * cw_r[...], axis=1, keepdims=True) + cb_r[...]
  h1o_r[...] = h1n
  c1o_r[...] = c1n
  h2o_r[...] = h2n
  c2o_r[...] = c2n


def _dense(obs, p_action, p_reward, h1, c1, h2, c2,
           enc_W1, enc_b1, enc_W2, enc_b2,
           Wih1, Whh1, bih1, bhh1, Wih2, Whh2, bih2, bhh2,
           actor_W, actor_b, critic_W, critic_b, m1, m2):
  out_shape = (
      jax.ShapeDtypeStruct((1, 6), jnp.float32),
      jax.ShapeDtypeStruct((1, 1), jnp.float32),
      jax.ShapeDtypeStruct((1, 128), jnp.float32),
      jax.ShapeDtypeStruct((1, 128), jnp.float32),
      jax.ShapeDtypeStruct((1, 64), jnp.float32),
      jax.ShapeDtypeStruct((1, 64), jnp.float32),
  )
  return pl.pallas_call(_dense_body, out_shape=out_shape)(
      obs, p_action, p_reward, h1, c1, h2, c2,
      enc_W1, enc_b1.reshape(1, -1), enc_W2, enc_b2.reshape(1, -1),
      Wih1, Whh1, bih1.reshape(1, -1), bhh1.reshape(1, -1),
      Wih2, Whh2, bih2.reshape(1, -1), bhh2.reshape(1, -1),
      actor_W, actor_b.reshape(1, -1), critic_W, critic_b.reshape(1, -1),
      m1, m2)


def kernel(obs, p_action, p_reward, h1, c1, h2, c2, cue,
           enc_W1, enc_b1, enc_W2, enc_b2, keys1, vals1, keys2, vals2,
           Wih1, Whh1, bih1, bhh1, Wih2, Whh2, bih2, bhh2,
           actor_W, actor_b, critic_W, critic_b):
  m1, m2 = _sc_retrieve(cue, keys1, vals1, keys2, vals2)
  lo, vo, h1o, c1o, h2o, c2o = _dense(
      obs, p_action, p_reward, h1[0], c1[0], h2[0], c2[0],
      enc_W1, enc_b1, enc_W2, enc_b2,
      Wih1, Whh1, bih1, bhh1, Wih2, Whh2, bih2, bhh2,
      actor_W, actor_b, critic_W, critic_b, m1, m2)
  return (lo[:, None, :], vo[:, None, :], h1o[None], c1o[None],
          h2o[None], c2o[None])


# speedup vs baseline: 4.6695x; 1.2630x over previous
import functools

import jax
import jax.numpy as jnp
from jax import lax
from jax.experimental import pallas as pl
from jax.experimental.pallas import tpu as pltpu
from jax.experimental.pallas import tpu_sc as plsc

_ROWS = 100000
_KD = 128
_H2 = 64
_CH = 128
_NCH = 49
_TILE_ROWS = 6256
_CLAMP = _ROWS - _CH
_NBUF = 3
_BIG = 3.0e38
_IMAX = 2**31 - 1


def _vgather(vec, idx):
  dnums = lax.GatherDimensionNumbers(
      offset_dims=(), collapsed_slice_dims=(0,), start_index_map=(0,))
  return lax.gather(vec, idx[:, None], dnums, (1,),
                    mode=lax.GatherScatterMode.PROMISE_IN_BOUNDS)


def _sc_retrieve(cue, keys1, vals1, keys2, vals2):
  mesh = plsc.VectorSubcoreMesh(core_axis_name="c", subcore_axis_name="s",
                                num_cores=2, num_subcores=16)

  def body(cue_hbm, keys1_hbm, vals1_hbm, keys2_hbm, vals2_hbm,
           m1_hbm, m2_hbm,
           b0, b1, b2, cue_v, stage_d, stage_i, alld, alli,
           shd, shi, sem0, sem1, sem2):
    cid = lax.axis_index("c")
    tid = lax.axis_index("s")
    iota16 = lax.iota(jnp.int32, 16)
    bufs = (b0, b1, b2)
    sems = (sem0, sem1, sem2)

    pltpu.sync_copy(cue_hbm.at[0], cue_v.at[pl.ds(0, _KD)])
    cue_v[pl.ds(_KD, 16)] = cue_v[pl.ds(0, 16)]

    base = tid * _TILE_ROWS
    rows_g = tuple(iota16 + g * 16 for g in range(8))

    def scan_dict(keys_hbm, vals_hbm, out_hbm):
      def start_chunk(k, b):
        st = jnp.minimum(base + k * _CH, _CLAMP)
        pltpu.make_async_copy(
            keys_hbm.at[pl.ds(st, _CH)], bufs[b], sems[b]).start()

      def wait_chunk(b):
        pltpu.make_async_copy(
            keys_hbm.at[pl.ds(0, _CH)], bufs[b], sems[b]).wait()

      for b in range(_NBUF):
        start_chunk(jnp.int32(b), b)

      def do_chunk(k, b, minv, mini):
        wait_chunk(b)
        st = jnp.minimum(base + k * _CH, _CLAMP)
        accs = tuple(jnp.zeros((16,), jnp.float32) for _ in range(8))

        def s_body(s, carry, b=b):
          jpos = (iota16 + s) & (_KD - 1)
          cv = cue_v[pl.ds(s, 16)]
          accs_t = carry
          out = []
          for g in range(8):
            kv = plsc.load_gather(bufs[b], (rows_g[g], jpos))
            t = kv - cv
            out.append(accs_t[g] + t * t)
          return tuple(out)

        accs = lax.fori_loop(0, _KD, s_body, accs)
        for g in range(8):
          rows = st + g * 16 + iota16
          d = accs[g]
          better = (d < minv) | ((d == minv) & (rows < mini))
          minv = jnp.where(better, d, minv)
          mini = jnp.where(better, rows, mini)
        return minv, mini

      minv = jnp.full((16,), _BIG, jnp.float32)
      mini = jnp.zeros((16,), jnp.int32)

      def outer(c, carry):
        mv, mi = carry
        for b in range(_NBUF):
          k = c * _NBUF + b
          mv, mi = do_chunk(k, b, mv, mi)

          @pl.when(k + _NBUF < _NCH)
          def _(k=k, b=b):
            start_chunk(k + _NBUF, b)
        return mv, mi

      minv, mini = lax.fori_loop(0, (_NCH - 1) // _NBUF, outer, (minv, mini))
      minv, mini = do_chunk(jnp.int32(_NCH - 1), 0, minv, mini)

      stage_d[...] = minv
      stage_i[...] = mini
      pltpu.sync_copy(stage_d, shd.at[pl.ds(tid * 16, 16)])
      pltpu.sync_copy(stage_i, shi.at[pl.ds(tid * 16, 16)])
      plsc.subcore_barrier()

      @pl.when(tid == 0)
      def _():
        pltpu.sync_copy(shd, alld)
        pltpu.sync_copy(shi, alli)
        mv = jnp.full((16,), _BIG, jnp.float32)
        mi = jnp.full((16,), _IMAX, jnp.int32)
        for t in range(16):
          d = alld[pl.ds(t * 16, 16)]
          ii = alli[pl.ds(t * 16, 16)]
          better = (d < mv) | ((d == mv) & (ii < mi))
          mv = jnp.where(better, d, mv)
          mi = jnp.where(better, ii, mi)
        m = jnp.min(mv)
        cand = jnp.where(mv == m, mi, jnp.full((16,), _IMAX, jnp.int32))
        bi = jnp.min(cand)
        pltpu.sync_copy(vals_hbm.at[pl.ds(bi, 1)], out_hbm)

    @pl.when(cid == 0)
    def _():
      scan_dict(keys1_hbm, vals1_hbm, m1_hbm)

    @pl.when(cid == 1)
    def _():
      scan_dict(keys2_hbm, vals2_hbm, m2_hbm)

  f = pl.kernel(
      body,
      out_type=(jax.ShapeDtypeStruct((1, _KD), jnp.float32),
                jax.ShapeDtypeStruct((1, _H2), jnp.float32)),
      mesh=mesh,
      compiler_params=pltpu.CompilerParams(needs_layout_passes=False),
      scratch_types=[
          pltpu.VMEM((_CH, _KD), jnp.float32),
          pltpu.VMEM((_CH, _KD), jnp.float32),
          pltpu.VMEM((_CH, _KD), jnp.float32),
          pltpu.VMEM((_KD + 16,), jnp.float32),
          pltpu.VMEM((16,), jnp.float32),
          pltpu.VMEM((16,), jnp.int32),
          pltpu.VMEM((256,), jnp.float32),
          pltpu.VMEM((256,), jnp.int32),
          pltpu.VMEM_SHARED((256,), jnp.float32),
          pltpu.VMEM_SHARED((256,), jnp.int32),
          pltpu.SemaphoreType.DMA,
          pltpu.SemaphoreType.DMA,
          pltpu.SemaphoreType.DMA,
      ],
  )
  return f(cue, keys1, vals1, keys2, vals2)


def _sig(x):
  return 1.0 / (1.0 + jnp.exp(-x))


def _dense_body(obs_r, pa_r, pr_r, h1_r, c1_r, h2_r, c2_r,
                w1_r, be1_r, w2_r, be2_r,
                wih1_r, whh1_r, bi1_r, bh1_r,
                wih2_r, whh2_r, bi2_r, bh2_r,
                aw_r, ab_r, cw_r, cb_r, m1_r, m2_r,
                lo_r, vo_r, h1o_r, c1o_r, h2o_r, c2o_r):
  def mmT(x, w):
    return lax.dot_general(x, w, (((1,), (1,)), ((), ())),
                           preferred_element_type=jnp.float32)

  obs_v = obs_r[...]
  f1 = jnp.maximum(mmT(obs_v, w1_r[...]) + be1_r[...], 0.0)
  feats = jnp.maximum(mmT(f1, w2_r[...]) + be2_r[...], 0.0)

  wih1 = wih1_r[...]
  h1v = h1_r[...]
  c1v = c1_r[...]
  g = (mmT(feats, wih1[:, 0:128]) + mmT(pr_r[...], wih1[:, 128:129]) +
       mmT(h1v, whh1_r[...]) + bi1_r[...] + bh1_r[...])
  i_g = g[:, 0:128]
  f_g = g[:, 128:256]
  g_g = g[:, 256:384]
  o_g = g[:, 384:512]
  r_g = g[:, 512:640]
  c1n = _sig(f_g) * c1v + _sig(i_g) * jnp.tanh(g_g) + _sig(r_g) * m1_r[...]
  h1n = _sig(o_g) * jnp.tanh(c1n)

  wih2 = wih2_r[...]
  h2v = h2_r[...]
  c2v = c2_r[...]
  g2 = (mmT(h1n, wih2[:, 0:128]) + mmT(feats, wih2[:, 128:256]) +
        mmT(pa_r[...], wih2[:, 256:262]) + mmT(h2v, whh2_r[...]) +
        bi2_r[...] + bh2_r[...])
  i2 = g2[:, 0:64]
  f2 = g2[:, 64:128]
  gg2 = g2[:, 128:192]
  o2 = g2[:, 192:256]
  r2 = g2[:, 256:320]
  c2n = _sig(f2) * c2v + _sig(i2) * jnp.tanh(gg2) + _sig(r2) * m2_r[...]
  h2n = _sig(o2) * jnp.tanh(c2n)

  lo_r[...] = mmT(h2n, aw_r[...]) + ab_r[...]
  vo_r[...] = jnp.sum(h2n * cw_r[...], axis=1, keepdims=True) + cb_r[...]
  h1o_r[...] = h1n
  c1o_r[...] = c1n
  h2o_r[...] = h2n
  c2o_r[...] = c2n


def _dense(obs, p_action, p_reward, h1, c1, h2, c2,
           enc_W1, enc_b1, enc_W2, enc_b2,
           Wih1, Whh1, bih1, bhh1, Wih2, Whh2, bih2, bhh2,
           actor_W, actor_b, critic_W, critic_b, m1, m2):
  out_shape = (
      jax.ShapeDtypeStruct((1, 6), jnp.float32),
      jax.ShapeDtypeStruct((1, 1), jnp.float32),
      jax.ShapeDtypeStruct((1, 128), jnp.float32),
      jax.ShapeDtypeStruct((1, 128), jnp.float32),
      jax.ShapeDtypeStruct((1, 64), jnp.float32),
      jax.ShapeDtypeStruct((1, 64), jnp.float32),
  )
  return pl.pallas_call(_dense_body, out_shape=out_shape)(
      obs, p_action, p_reward, h1, c1, h2, c2,
      enc_W1, enc_b1.reshape(1, -1), enc_W2, enc_b2.reshape(1, -1),
      Wih1, Whh1, bih1.reshape(1, -1), bhh1.reshape(1, -1),
      Wih2, Whh2, bih2.reshape(1, -1), bhh2.reshape(1, -1),
      actor_W, actor_b.reshape(1, -1), critic_W, critic_b.reshape(1, -1),
      m1, m2)


def kernel(obs, p_action, p_reward, h1, c1, h2, c2, cue,
           enc_W1, enc_b1, enc_W2, enc_b2, keys1, vals1, keys2, vals2,
           Wih1, Whh1, bih1, bhh1, Wih2, Whh2, bih2, bhh2,
           actor_W, actor_b, critic_W, critic_b):
  m1, m2 = _sc_retrieve(cue, keys1, vals1, keys2, vals2)
  lo, vo, h1o, c1o, h2o, c2o = _dense(
      obs, p_action, p_reward, h1[0], c1[0], h2[0], c2[0],
      enc_W1, enc_b1, enc_W2, enc_b2,
      Wih1, Whh1, bih1, bhh1, Wih2, Whh2, bih2, bhh2,
      actor_W, actor_b, critic_W, critic_b, m1, m2)
  return (lo[:, None, :], vo[:, None, :], h1o[None], c1o[None],
          h2o[None], c2o[None])


# speedup vs baseline: 4.9695x; 1.0642x over previous
import functools

import jax
import jax.numpy as jnp
from jax import lax
from jax.experimental import pallas as pl
from jax.experimental.pallas import tpu as pltpu
from jax.experimental.pallas import tpu_sc as plsc

_ROWS = 100000
_KD = 128
_H2 = 64
_CH = 128
_NCH = 49
_TILE_ROWS = 6256
_CLAMP = _ROWS - _CH
_NBUF = 3
_BIG = 3.0e38
_IMAX = 2**31 - 1


def _vgather(vec, idx):
  dnums = lax.GatherDimensionNumbers(
      offset_dims=(), collapsed_slice_dims=(0,), start_index_map=(0,))
  return lax.gather(vec, idx[:, None], dnums, (1,),
                    mode=lax.GatherScatterMode.PROMISE_IN_BOUNDS)


def _sc_retrieve(cue, keys1, vals1, keys2, vals2):
  mesh = plsc.VectorSubcoreMesh(core_axis_name="c", subcore_axis_name="s",
                                num_cores=2, num_subcores=16)

  def body(cue_hbm, keys1_hbm, vals1_hbm, keys2_hbm, vals2_hbm,
           m1_hbm, m2_hbm,
           b0, b1, b2, cue_v, stage_d, stage_i, alld, alli,
           shd, shi, sem0, sem1, sem2):
    cid = lax.axis_index("c")
    tid = lax.axis_index("s")
    iota16 = lax.iota(jnp.int32, 16)
    bufs = (b0, b1, b2)
    sems = (sem0, sem1, sem2)

    pltpu.sync_copy(cue_hbm.at[0], cue_v.at[pl.ds(0, _KD)])
    cue_v[pl.ds(_KD, 16)] = cue_v[pl.ds(0, 16)]

    base = tid * _TILE_ROWS
    rows_g = tuple(iota16 + g * 16 for g in range(8))

    def scan_dict(keys_hbm, vals_hbm, out_hbm):
      def start_chunk(k, b):
        st = jnp.minimum(base + k * _CH, _CLAMP)
        pltpu.make_async_copy(
            keys_hbm.at[pl.ds(st, _CH)], bufs[b], sems[b]).start()

      def wait_chunk(b):
        pltpu.make_async_copy(
            keys_hbm.at[pl.ds(0, _CH)], bufs[b], sems[b]).wait()

      for b in range(_NBUF):
        start_chunk(jnp.int32(b), b)

      def do_chunk(k, b, minv, mini):
        wait_chunk(b)
        st = jnp.minimum(base + k * _CH, _CLAMP)
        accs = tuple(jnp.zeros((16,), jnp.float32) for _ in range(8))

        def s_body(s, carry, b=b):
          jpos = (iota16 + s) & (_KD - 1)
          cv = cue_v[pl.ds(s, 16)]
          accs_t = carry
          out = []
          for g in range(8):
            kv = plsc.load_gather(bufs[b], (rows_g[g], jpos))
            t = kv - cv
            out.append(accs_t[g] + t * t)
          return tuple(out)

        accs = plsc.parallel_loop(0, _KD, unroll=4, carry=accs)(s_body)
        for g in range(8):
          rows = st + g * 16 + iota16
          d = accs[g]
          better = (d < minv) | ((d == minv) & (rows < mini))
          minv = jnp.where(better, d, minv)
          mini = jnp.where(better, rows, mini)
        return minv, mini

      minv = jnp.full((16,), _BIG, jnp.float32)
      mini = jnp.zeros((16,), jnp.int32)

      def outer(c, carry):
        mv, mi = carry
        for b in range(_NBUF):
          k = c * _NBUF + b
          mv, mi = do_chunk(k, b, mv, mi)

          @pl.when(k + _NBUF < _NCH)
          def _(k=k, b=b):
            start_chunk(k + _NBUF, b)
        return mv, mi

      minv, mini = lax.fori_loop(0, (_NCH - 1) // _NBUF, outer, (minv, mini))
      minv, mini = do_chunk(jnp.int32(_NCH - 1), 0, minv, mini)

      stage_d[...] = minv
      stage_i[...] = mini
      pltpu.sync_copy(stage_d, shd.at[pl.ds(tid * 16, 16)])
      pltpu.sync_copy(stage_i, shi.at[pl.ds(tid * 16, 16)])
      plsc.subcore_barrier()

      @pl.when(tid == 0)
      def _():
        pltpu.sync_copy(shd, alld)
        pltpu.sync_copy(shi, alli)
        mv = jnp.full((16,), _BIG, jnp.float32)
        mi = jnp.full((16,), _IMAX, jnp.int32)
        for t in range(16):
          d = alld[pl.ds(t * 16, 16)]
          ii = alli[pl.ds(t * 16, 16)]
          better = (d < mv) | ((d == mv) & (ii < mi))
          mv = jnp.where(better, d, mv)
          mi = jnp.where(better, ii, mi)
        m = jnp.min(mv)
        cand = jnp.where(mv == m, mi, jnp.full((16,), _IMAX, jnp.int32))
        bi = jnp.min(cand)
        pltpu.sync_copy(vals_hbm.at[pl.ds(bi, 1)], out_hbm)

    @pl.when(cid == 0)
    def _():
      scan_dict(keys1_hbm, vals1_hbm, m1_hbm)

    @pl.when(cid == 1)
    def _():
      scan_dict(keys2_hbm, vals2_hbm, m2_hbm)

  f = pl.kernel(
      body,
      out_type=(jax.ShapeDtypeStruct((1, _KD), jnp.float32),
                jax.ShapeDtypeStruct((1, _H2), jnp.float32)),
      mesh=mesh,
      compiler_params=pltpu.CompilerParams(needs_layout_passes=False),
      scratch_types=[
          pltpu.VMEM((_CH, _KD), jnp.float32),
          pltpu.VMEM((_CH, _KD), jnp.float32),
          pltpu.VMEM((_CH, _KD), jnp.float32),
          pltpu.VMEM((_KD + 16,), jnp.float32),
          pltpu.VMEM((16,), jnp.float32),
          pltpu.VMEM((16,), jnp.int32),
          pltpu.VMEM((256,), jnp.float32),
          pltpu.VMEM((256,), jnp.int32),
          pltpu.VMEM_SHARED((256,), jnp.float32),
          pltpu.VMEM_SHARED((256,), jnp.int32),
          pltpu.SemaphoreType.DMA,
          pltpu.SemaphoreType.DMA,
          pltpu.SemaphoreType.DMA,
      ],
  )
  return f(cue, keys1, vals1, keys2, vals2)


def _sig(x):
  return 1.0 / (1.0 + jnp.exp(-x))


def _dense_body(obs_r, pa_r, pr_r, h1_r, c1_r, h2_r, c2_r,
                w1_r, be1_r, w2_r, be2_r,
                wih1_r, whh1_r, bi1_r, bh1_r,
                wih2_r, whh2_r, bi2_r, bh2_r,
                aw_r, ab_r, cw_r, cb_r, m1_r, m2_r,
                lo_r, vo_r, h1o_r, c1o_r, h2o_r, c2o_r):
  def mmT(x, w):
    return lax.dot_general(x, w, (((1,), (1,)), ((), ())),
                           preferred_element_type=jnp.float32)

  obs_v = obs_r[...]
  f1 = jnp.maximum(mmT(obs_v, w1_r[...]) + be1_r[...], 0.0)
  feats = jnp.maximum(mmT(f1, w2_r[...]) + be2_r[...], 0.0)

  wih1 = wih1_r[...]
  h1v = h1_r[...]
  c1v = c1_r[...]
  g = (mmT(feats, wih1[:, 0:128]) + mmT(pr_r[...], wih1[:, 128:129]) +
       mmT(h1v, whh1_r[...]) + bi1_r[...] + bh1_r[...])
  i_g = g[:, 0:128]
  f_g = g[:, 128:256]
  g_g = g[:, 256:384]
  o_g = g[:, 384:512]
  r_g = g[:, 512:640]
  c1n = _sig(f_g) * c1v + _sig(i_g) * jnp.tanh(g_g) + _sig(r_g) * m1_r[...]
  h1n = _sig(o_g) * jnp.tanh(c1n)

  wih2 = wih2_r[...]
  h2v = h2_r[...]
  c2v = c2_r[...]
  g2 = (mmT(h1n, wih2[:, 0:128]) + mmT(feats, wih2[:, 128:256]) +
        mmT(pa_r[...], wih2[:, 256:262]) + mmT(h2v, whh2_r[...]) +
        bi2_r[...] + bh2_r[...])
  i2 = g2[:, 0:64]
  f2 = g2[:, 64:128]
  gg2 = g2[:, 128:192]
  o2 = g2[:, 192:256]
  r2 = g2[:, 256:320]
  c2n = _sig(f2) * c2v + _sig(i2) * jnp.tanh(gg2) + _sig(r2) * m2_r[...]
  h2n = _sig(o2) * jnp.tanh(c2n)

  lo_r[...] = mmT(h2n, aw_r[...]) + ab_r[...]
  vo_r[...] = jnp.sum(h2n * cw_r[...], axis=1, keepdims=True) + cb_r[...]
  h1o_r[...] = h1n
  c1o_r[...] = c1n
  h2o_r[...] = h2n
  c2o_r[...] = c2n


def _dense(obs, p_action, p_reward, h1, c1, h2, c2,
           enc_W1, enc_b1, enc_W2, enc_b2,
           Wih1, Whh1, bih1, bhh1, Wih2, Whh2, bih2, bhh2,
           actor_W, actor_b, critic_W, critic_b, m1, m2):
  out_shape = (
      jax.ShapeDtypeStruct((1, 6), jnp.float32),
      jax.ShapeDtypeStruct((1, 1), jnp.float32),
      jax.ShapeDtypeStruct((1, 128), jnp.float32),
      jax.ShapeDtypeStruct((1, 128), jnp.float32),
      jax.ShapeDtypeStruct((1, 64), jnp.float32),
      jax.ShapeDtypeStruct((1, 64), jnp.float32),
  )
  return pl.pallas_call(_dense_body, out_shape=out_shape)(
      obs, p_action, p_reward, h1, c1, h2, c2,
      enc_W1, enc_b1.reshape(1, -1), enc_W2, enc_b2.reshape(1, -1),
      Wih1, Whh1, bih1.reshape(1, -1), bhh1.reshape(1, -1),
      Wih2, Whh2, bih2.reshape(1, -1), bhh2.reshape(1, -1),
      actor_W, actor_b.reshape(1, -1), critic_W, critic_b.reshape(1, -1),
      m1, m2)


def kernel(obs, p_action, p_reward, h1, c1, h2, c2, cue,
           enc_W1, enc_b1, enc_W2, enc_b2, keys1, vals1, keys2, vals2,
           Wih1, Whh1, bih1, bhh1, Wih2, Whh2, bih2, bhh2,
           actor_W, actor_b, critic_W, critic_b):
  m1, m2 = _sc_retrieve(cue, keys1, vals1, keys2, vals2)
  lo, vo, h1o, c1o, h2o, c2o = _dense(
      obs, p_action, p_reward, h1[0], c1[0], h2[0], c2[0],
      enc_W1, enc_b1, enc_W2, enc_b2,
      Wih1, Whh1, bih1, bhh1, Wih2, Whh2, bih2, bhh2,
      actor_W, actor_b, critic_W, critic_b, m1, m2)
  return (lo[:, None, :], vo[:, None, :], h1o[None], c1o[None],
          h2o[None], c2o[None])
